# baseline (speedup 1.0000x reference)
import functools

import jax
import jax.numpy as jnp
from jax import lax
from jax.experimental import pallas as pl
from jax.experimental.pallas import tpu as pltpu
from jax.experimental.pallas import tpu_sc as plsc

B = 16384
D = 32
NC = 2
NS = 16
NW = NC * NS
BPW = B // NW
L = 16


def _sc_body(user_hbm, movie_hbm, uf_hbm, mf_hbm, out_hbm,
             uidx, midx, urows, mrows, outv, su, sm):
    wid = lax.axis_index("s") * NC + lax.axis_index("c")
    base = wid * BPW
    pltpu.sync_copy(user_hbm.at[pl.ds(base, BPW)], uidx)
    pltpu.sync_copy(movie_hbm.at[pl.ds(base, BPW)], midx)

    cu = pltpu.async_copy(uf_hbm.at[uidx], urows, su)
    cm = pltpu.async_copy(mf_hbm.at[midx], mrows, sm)
    cu.wait()
    cm.wait()

    lane = lax.iota(jnp.int32, L)

    @pl.loop(0, BPW, step=L)
    def _(i):
        acc = jnp.zeros((L,), jnp.float32)
        for k in range(L):
            u = urows[i + k, pl.ds(0, D)]
            m = mrows[i + k, pl.ds(0, D)]
            u0, u1 = plsc.unpack(u, format=plsc.PackFormat.INTERLEAVED)
            m0, m1 = plsc.unpack(m, format=plsc.PackFormat.INTERLEAVED)
            s = jnp.sum(u0 * m0 + u1 * m1)
            acc = jnp.where(lane == k, s, acc)
        outv[pl.ds(i, L)] = acc

    pltpu.sync_copy(outv, out_hbm.at[pl.ds(base, BPW)])


def kernel(user, movie, user_factors, movie_factors):
    mesh = plsc.VectorSubcoreMesh(core_axis_name="c", subcore_axis_name="s")
    kern = pl.kernel(
        _sc_body,
        out_type=jax.ShapeDtypeStruct((B,), jnp.float32),
        mesh=mesh,
        compiler_params=pltpu.CompilerParams(use_tc_tiling_on_sc=False,
                                             needs_layout_passes=False),
        scratch_types=[
            pltpu.VMEM((BPW,), jnp.int32),
            pltpu.VMEM((BPW,), jnp.int32),
            pltpu.VMEM((BPW, D), jnp.bfloat16),
            pltpu.VMEM((BPW, D), jnp.bfloat16),
            pltpu.VMEM((BPW,), jnp.float32),
            pltpu.SemaphoreType.DMA,
            pltpu.SemaphoreType.DMA,
        ],
    )
    return kern(user.astype(jnp.int32), movie.astype(jnp.int32),
                user_factors.astype(jnp.bfloat16),
                movie_factors.astype(jnp.bfloat16))


# speedup vs baseline: 1.1936x; 1.1936x over previous
import jax
import jax.numpy as jnp
from jax import lax
from jax.experimental import pallas as pl
from jax.experimental.pallas import tpu as pltpu
from jax.experimental.pallas import tpu_sc as plsc

B = 16384
D = 32
NC = 2
NS = 16
NW = NC * NS
BPW = B // NW
L = 16


def _sc_body(user_hbm, movie_hbm, uf_hbm, mf_hbm, out_hbm,
             uidx, midx, urows, mrows, outv, su, sm):
    wid = lax.axis_index("s") * NC + lax.axis_index("c")
    base = wid * BPW
    pltpu.sync_copy(user_hbm.at[pl.ds(base, BPW)], uidx)
    pltpu.sync_copy(movie_hbm.at[pl.ds(base, BPW)], midx)

    cu = pltpu.async_copy(uf_hbm.at[uidx], urows, su)
    cm = pltpu.async_copy(mf_hbm.at[midx], mrows, sm)
    cu.wait()
    cm.wait()

    lane = lax.iota(jnp.int32, L)

    @pl.loop(0, BPW, step=L)
    def _(i):
        acc = jnp.zeros((L,), jnp.float32)
        for k in range(L):
            u0 = urows[i + k, pl.ds(0, L)]
            u1 = urows[i + k, pl.ds(L, L)]
            m0 = mrows[i + k, pl.ds(0, L)]
            m1 = mrows[i + k, pl.ds(L, L)]
            s = jnp.sum(u0 * m0 + u1 * m1)
            acc = jnp.where(lane == k, s, acc)
        outv[pl.ds(i, L)] = acc

    pltpu.sync_copy(outv, out_hbm.at[pl.ds(base, BPW)])


def kernel(user, movie, user_factors, movie_factors):
    mesh = plsc.VectorSubcoreMesh(core_axis_name="c", subcore_axis_name="s")
    kern = pl.kernel(
        _sc_body,
        out_type=jax.ShapeDtypeStruct((B,), jnp.float32),
        mesh=mesh,
        compiler_params=pltpu.CompilerParams(use_tc_tiling_on_sc=False,
                                             needs_layout_passes=False),
        scratch_types=[
            pltpu.VMEM((BPW,), jnp.int32),
            pltpu.VMEM((BPW,), jnp.int32),
            pltpu.VMEM((BPW, D), jnp.float32),
            pltpu.VMEM((BPW, D), jnp.float32),
            pltpu.VMEM((BPW,), jnp.float32),
            pltpu.SemaphoreType.DMA,
            pltpu.SemaphoreType.DMA,
        ],
    )
    return kern(user.astype(jnp.int32), movie.astype(jnp.int32),
                user_factors, movie_factors)
